# baseline (speedup 1.0000x reference)
import functools
import math

import jax
import jax.numpy as jnp
from jax.experimental import pallas as pl
from jax.experimental.pallas import tpu as pltpu

TAU = 1.5
LN_EPS = 1e-5
TB = 2048
CHUNK = 512


def _norm(x):
    mu = jnp.mean(x, axis=-1, keepdims=True)
    ms = jnp.mean(x * x, axis=-1, keepdims=True)
    return (x - mu) * jax.lax.rsqrt(ms - mu * mu + LN_EPS)


def _body(x_ref, w1_ref, b1_ref, w2_ref, b2_ref, p_ref, bias_ref, psum_ref,
          w1sum_ref, s_ref, logits_ref, usage_ref, ent_ref, *, n_tokens):
    tb = x_ref.shape[0]
    chunk = CHUNK if tb % CHUNK == 0 else tb
    usage_parts = []
    ent_parts = []
    for c in range(tb // chunk):
        rows = slice(c * chunk, (c + 1) * chunk)
        x = x_ref[rows, :]
        mu1 = jnp.mean(x, axis=-1, keepdims=True)
        ms1 = jnp.mean(x * x, axis=-1, keepdims=True)
        r1 = jax.lax.rsqrt(ms1 - mu1 * mu1 + LN_EPS)
        xw = jax.lax.dot_general(x, w1_ref[...], (((1,), (1,)), ((), ())),
                                 preferred_element_type=jnp.float32)
        h = r1 * xw - (r1 * mu1) * w1sum_ref[...] + b1_ref[...]
        h = 0.5 * h * (1.0 + jax.lax.erf(h * (1.0 / math.sqrt(2.0))))
        y = jax.lax.dot_general(h, w2_ref[...], (((1,), (1,)), ((), ())),
                                preferred_element_type=jnp.float32)
        y = y + b2_ref[...]
        mu = jnp.mean(y, axis=-1, keepdims=True)
        ms = jnp.mean(y * y, axis=-1, keepdims=True)
        r = jax.lax.rsqrt(ms - mu * mu + LN_EPS)
        yp = jax.lax.dot_general(y, p_ref[...], (((1,), (1,)), ((), ())),
                                 preferred_element_type=jnp.float32)
        logits = r * yp - (r * mu) * psum_ref[...] + bias_ref[...]
        logits_ref[rows, :] = logits

        z = logits * (1.0 / TAU)
        e = jnp.exp(z)
        denom = jnp.sum(e, axis=-1, keepdims=True)
        s = e * (1.0 / denom)
        s_ref[rows, :] = s

        usage_parts.append(jnp.sum(s, axis=0, keepdims=True))
        ent_parts.append(jnp.sum(jnp.log(denom[:, 0]) -
                                 jnp.sum(s * z, axis=-1)))
    usage_ref[...] = (sum(usage_parts) * (1.0 / n_tokens)).reshape(1, 1, -1)
    ent_ref[...] = (sum(ent_parts) * (1.0 / n_tokens)).reshape(1, 1, 1)


def kernel(H, pre_g, pre_b, W1, b1, W2, b2, post_g, post_b, P, bias):
    b_, v_, d_ = H.shape
    k_ = P.shape[0]
    hid = W1.shape[0]
    t = b_ * v_
    tb = TB if t % TB == 0 else t
    x = H.reshape(t, d_)

    w1f = W1 * pre_g[None, :]
    b1f = b1 + W1 @ pre_b
    scale = 1.0 / math.sqrt(d_)
    pf = P * (post_g[None, :] * scale)
    biasf = bias + scale * (P @ post_b)

    body = functools.partial(_body, n_tokens=t)
    full = lambda shape: pl.BlockSpec(shape, lambda i: (0, 0))
    s_flat, logits_flat, usage, ent = pl.pallas_call(
        body,
        grid=(t // tb,),
        in_specs=[
            pl.BlockSpec((tb, d_), lambda i: (i, 0)),
            full((hid, d_)), full((1, hid)),
            full((d_, hid)), full((1, d_)),
            full((k_, d_)), full((1, k_)), full((1, k_)),
            full((1, hid)),
        ],
        out_specs=[
            pl.BlockSpec((tb, k_), lambda i: (i, 0)),
            pl.BlockSpec((tb, k_), lambda i: (i, 0)),
            pl.BlockSpec((1, 1, k_), lambda i: (i, 0, 0)),
            pl.BlockSpec((1, 1, 1), lambda i: (i, 0, 0)),
        ],
        out_shape=[
            jax.ShapeDtypeStruct((t, k_), jnp.float32),
            jax.ShapeDtypeStruct((t, k_), jnp.float32),
            jax.ShapeDtypeStruct((t // tb, 1, k_), jnp.float32),
            jax.ShapeDtypeStruct((t // tb, 1, 1), jnp.float32),
        ],
        compiler_params=pltpu.CompilerParams(
            vmem_limit_bytes=128 * 1024 * 1024,
            dimension_semantics=("parallel",),
        ),
    )(x, w1f, b1f.reshape(1, hid), W2, b2.reshape(1, d_),
      pf, biasf.reshape(1, k_), jnp.sum(pf, axis=1).reshape(1, k_),
      jnp.sum(w1f, axis=1).reshape(1, hid))

    s = s_flat.reshape(b_, v_, k_)
    logits = logits_flat.reshape(b_, v_, k_)
    return (s, s, logits, jnp.sum(usage, axis=(0, 1)),
            jnp.sum(ent, axis=(0, 1)))


# speedup vs baseline: 1.0224x; 1.0224x over previous
import functools
import math

import jax
import jax.numpy as jnp
from jax.experimental import pallas as pl
from jax.experimental.pallas import tpu as pltpu

TAU = 1.5
LN_EPS = 1e-5
TB = 2048
CHUNK = 1024


def _norm(x):
    mu = jnp.mean(x, axis=-1, keepdims=True)
    ms = jnp.mean(x * x, axis=-1, keepdims=True)
    return (x - mu) * jax.lax.rsqrt(ms - mu * mu + LN_EPS)


def _body(x_ref, w1_ref, b1_ref, w2_ref, b2_ref, p_ref, bias_ref, psum_ref,
          s_ref, logits_ref, usage_ref, ent_ref, *, n_tokens):
    tb = x_ref.shape[0]
    chunk = CHUNK if tb % CHUNK == 0 else tb
    usage_parts = []
    ent_parts = []
    for c in range(tb // chunk):
        rows = slice(c * chunk, (c + 1) * chunk)
        xn = _norm(x_ref[rows, :])
        h = jax.lax.dot_general(xn, w1_ref[...], (((1,), (1,)), ((), ())),
                                preferred_element_type=jnp.float32)
        h = h + b1_ref[...]
        h = 0.5 * h * (1.0 + jax.lax.erf(h * (1.0 / math.sqrt(2.0))))
        y = jax.lax.dot_general(h, w2_ref[...], (((1,), (1,)), ((), ())),
                                preferred_element_type=jnp.float32)
        y = y + b2_ref[...]
        mu = jnp.mean(y, axis=-1, keepdims=True)
        ms = jnp.mean(y * y, axis=-1, keepdims=True)
        r = jax.lax.rsqrt(ms - mu * mu + LN_EPS)
        yp = jax.lax.dot_general(y, p_ref[...], (((1,), (1,)), ((), ())),
                                 preferred_element_type=jnp.float32)
        logits = r * yp - (r * mu) * psum_ref[...] + bias_ref[...]
        logits_ref[rows, :] = logits

        z = logits * (1.0 / TAU)
        e = jnp.exp(z)
        denom = jnp.sum(e, axis=-1, keepdims=True)
        s = e * (1.0 / denom)
        s_ref[rows, :] = s

        usage_parts.append(jnp.sum(s, axis=0, keepdims=True))
        ent_parts.append(jnp.sum(jnp.log(denom[:, 0]) -
                                 jnp.sum(s * z, axis=-1)))
    usage_ref[...] = (sum(usage_parts) * (1.0 / n_tokens)).reshape(1, 1, -1)
    ent_ref[...] = (sum(ent_parts) * (1.0 / n_tokens)).reshape(1, 1, 1)


def kernel(H, pre_g, pre_b, W1, b1, W2, b2, post_g, post_b, P, bias):
    b_, v_, d_ = H.shape
    k_ = P.shape[0]
    hid = W1.shape[0]
    t = b_ * v_
    tb = TB if t % TB == 0 else t
    x = H.reshape(t, d_)

    w1f = W1 * pre_g[None, :]
    b1f = b1 + W1 @ pre_b
    scale = 1.0 / math.sqrt(d_)
    pf = P * (post_g[None, :] * scale)
    biasf = bias + scale * (P @ post_b)

    body = functools.partial(_body, n_tokens=t)
    full = lambda shape: pl.BlockSpec(shape, lambda i: (0, 0))
    s_flat, logits_flat, usage, ent = pl.pallas_call(
        body,
        grid=(t // tb,),
        in_specs=[
            pl.BlockSpec((tb, d_), lambda i: (i, 0)),
            full((hid, d_)), full((1, hid)),
            full((d_, hid)), full((1, d_)),
            full((k_, d_)), full((1, k_)), full((1, k_)),
        ],
        out_specs=[
            pl.BlockSpec((tb, k_), lambda i: (i, 0)),
            pl.BlockSpec((tb, k_), lambda i: (i, 0)),
            pl.BlockSpec((1, 1, k_), lambda i: (i, 0, 0)),
            pl.BlockSpec((1, 1, 1), lambda i: (i, 0, 0)),
        ],
        out_shape=[
            jax.ShapeDtypeStruct((t, k_), jnp.float32),
            jax.ShapeDtypeStruct((t, k_), jnp.float32),
            jax.ShapeDtypeStruct((t // tb, 1, k_), jnp.float32),
            jax.ShapeDtypeStruct((t // tb, 1, 1), jnp.float32),
        ],
        compiler_params=pltpu.CompilerParams(
            vmem_limit_bytes=128 * 1024 * 1024,
            dimension_semantics=("parallel",),
        ),
    )(x, w1f, b1f.reshape(1, hid), W2, b2.reshape(1, d_),
      pf, biasf.reshape(1, k_), jnp.sum(pf, axis=1).reshape(1, k_))

    s = s_flat.reshape(b_, v_, k_)
    logits = logits_flat.reshape(b_, v_, k_)
    return (s, s, logits, jnp.sum(usage, axis=(0, 1)),
            jnp.sum(ent, axis=(0, 1)))
